# Initial kernel scaffold; baseline (speedup 1.0000x reference)
import jax
import jax.numpy as jnp
from jax.experimental import pallas as pl


def kernel(x, edge_index, W0, b0, W1, b1, a_mu, a_log_sigma, a_mu_first, a_log_sigma_first, eps_first, eps_rest):
    raise NotImplementedError("write your pallas kernel here")


# speedup vs baseline: 3.5879x; 3.5879x over previous
import functools
import math

import jax
import jax.numpy as jnp
from jax import lax
from jax.experimental import pallas as pl
from jax.experimental.pallas import tpu as pltpu
from jax.experimental.pallas import tpu_sc as plsc

N = 10000
E = 160000
D = 128

NC = 2
NS = 16
L = 16
NW = NC * NS
G = D // L
EPT = E // NW
CHUNK = 128
FULL_CHUNKS = EPT // CHUNK
TAIL = EPT - FULL_CHUNKS * CHUNK
RPT = 624
REM_ROWS = N - NS * RPT
LOG2PI = math.log(2.0 * math.pi)


def _sc_layer_body(src_hbm, dst_hbm, table_hbm, eps_hbm, mu_hbm, sig_hbm,
                   part_hbm, nll_hbm,
                   src_v, dst_v, rows_v, eps_v, srct_v, dstt_v, rowst_v,
                   epst_v, mu_v, sig_v, nll_v, hacc_sh, sem):
    c = lax.axis_index("c")
    s = lax.axis_index("s")
    wid = c * NS + s
    tile_base = wid * EPT

    pltpu.sync_copy(mu_hbm, mu_v)
    pltpu.sync_copy(sig_hbm, sig_v)
    mu_r = [mu_v[g, :] for g in range(G)]
    sig_r = [sig_v[g, :] for g in range(G)]

    def zero_row(r, carry):
        for g in range(G):
            rows_v[r, pl.ds(g * L, L)] = jnp.zeros((L,), jnp.float32)
        return carry
    lax.fori_loop(0, CHUNK, zero_row, 0)
    row0 = s * RPT
    for k in range(RPT // CHUNK):
        pltpu.sync_copy(rows_v, hacc_sh.at[pl.ds(row0 + k * CHUNK, CHUNK)])
    rem = RPT - (RPT // CHUNK) * CHUNK
    if rem:
        pltpu.sync_copy(rows_v.at[pl.ds(0, rem)],
                        hacc_sh.at[pl.ds(row0 + (RPT // CHUNK) * CHUNK, rem)])

    @pl.when(s == NS - 1)
    def _zero_leftover():
        pltpu.sync_copy(rows_v.at[pl.ds(0, REM_ROWS)],
                        hacc_sh.at[pl.ds(NS * RPT, REM_ROWS)])
    plsc.subcore_barrier()

    def scale_rows(rows_ref, eps_ref, cnt, acc):
        def row_body(r, acc):
            for g in range(G):
                sl = pl.ds(g * L, L)
                a = mu_r[g] + sig_r[g] * eps_ref[r, sl]
                t = a - 1.0
                acc = acc + t * t
                rows_ref[r, sl] = rows_ref[r, sl] * a
            return acc
        return lax.fori_loop(0, cnt, row_body, acc)

    def chunk_body(i, acc):
        base = tile_base + i * CHUNK
        pltpu.sync_copy(src_hbm.at[pl.ds(base, CHUNK)], src_v)
        pltpu.sync_copy(dst_hbm.at[pl.ds(base, CHUNK)], dst_v)
        pltpu.async_copy(table_hbm.at[src_v], rows_v, sem).wait()
        pltpu.sync_copy(eps_hbm.at[pl.ds(base, CHUNK)], eps_v)
        acc = scale_rows(rows_v, eps_v, CHUNK, acc)
        pltpu.sync_copy(rows_v, hacc_sh.at[dst_v], add=True)
        return acc

    acc = lax.fori_loop(0, FULL_CHUNKS, chunk_body,
                        jnp.zeros((L,), jnp.float32))

    tbase = tile_base + FULL_CHUNKS * CHUNK
    pltpu.sync_copy(src_hbm.at[pl.ds(tbase, TAIL)], srct_v)
    pltpu.sync_copy(dst_hbm.at[pl.ds(tbase, TAIL)], dstt_v)
    pltpu.async_copy(table_hbm.at[srct_v], rowst_v, sem).wait()
    pltpu.sync_copy(eps_hbm.at[pl.ds(tbase, TAIL)], epst_v)
    acc = scale_rows(rowst_v, epst_v, TAIL, acc)
    pltpu.sync_copy(rowst_v, hacc_sh.at[dstt_v], add=True)

    nll_v[...] = acc
    pltpu.sync_copy(nll_v, nll_hbm.at[pl.ds(wid * L, L)])

    plsc.subcore_barrier()
    pltpu.sync_copy(hacc_sh.at[pl.ds(row0, RPT)],
                    part_hbm.at[pl.ds(c * N + row0, RPT)])

    @pl.when(s == NS - 1)
    def _read_leftover():
        pltpu.sync_copy(hacc_sh.at[pl.ds(NS * RPT, REM_ROWS)],
                        part_hbm.at[pl.ds(c * N + NS * RPT, REM_ROWS)])


def _make_sc_layer():
    mesh = plsc.VectorSubcoreMesh(core_axis_name="c", subcore_axis_name="s",
                                  num_cores=NC, num_subcores=NS)
    return pl.kernel(
        _sc_layer_body,
        out_type=(
            jax.ShapeDtypeStruct((NC * N, D), jnp.float32),
            jax.ShapeDtypeStruct((NW * L,), jnp.float32),
        ),
        mesh=mesh,
        scratch_types=[
            pltpu.VMEM((CHUNK,), jnp.int32),
            pltpu.VMEM((CHUNK,), jnp.int32),
            pltpu.VMEM((CHUNK, D), jnp.float32),
            pltpu.VMEM((CHUNK, D), jnp.float32),
            pltpu.VMEM((TAIL,), jnp.int32),
            pltpu.VMEM((TAIL,), jnp.int32),
            pltpu.VMEM((TAIL, D), jnp.float32),
            pltpu.VMEM((TAIL, D), jnp.float32),
            pltpu.VMEM((G, L), jnp.float32),
            pltpu.VMEM((G, L), jnp.float32),
            pltpu.VMEM((L,), jnp.float32),
            pltpu.VMEM_SHARED((N, D), jnp.float32),
            pltpu.SemaphoreType.DMA,
        ],
    )


_ROWS_BLK = 1000
_GRID = N // _ROWS_BLK


def _tc_mid_body(p0_ref, p1_ref, w_ref, b_ref, o_ref):
    s = p0_ref[...] + p1_ref[...]
    y = jnp.dot(s, w_ref[...], preferred_element_type=jnp.float32)
    o_ref[...] = jnp.maximum(y + b_ref[...], 0.0)


def _tc_final_body(p0_ref, p1_ref, w_ref, b_ref, n1_ref, n2_ref,
                   o_ref, nll_ref):
    s = p0_ref[...] + p1_ref[...]
    y = jnp.dot(s, w_ref[...], preferred_element_type=jnp.float32)
    o_ref[...] = y + b_ref[...]

    @pl.when(pl.program_id(0) == 0)
    def _():
        tot = jnp.sum(n1_ref[...]) + jnp.sum(n2_ref[...])
        nll_ref[...] = jnp.reshape(0.5 * tot / float(E * D) + LOG2PI, (1, 1))


def _tc_mid(parts, w, b):
    return pl.pallas_call(
        _tc_mid_body,
        grid=(_GRID,),
        in_specs=[
            pl.BlockSpec((_ROWS_BLK, D), lambda i: (i, 0)),
            pl.BlockSpec((_ROWS_BLK, D), lambda i: (i + _GRID, 0)),
            pl.BlockSpec((D, D), lambda i: (0, 0)),
            pl.BlockSpec((1, D), lambda i: (0, 0)),
        ],
        out_specs=pl.BlockSpec((_ROWS_BLK, D), lambda i: (i, 0)),
        out_shape=jax.ShapeDtypeStruct((N, D), jnp.float32),
    )(parts, parts, w, b)


def _tc_final(parts, w, b, n1, n2):
    return pl.pallas_call(
        _tc_final_body,
        grid=(_GRID,),
        in_specs=[
            pl.BlockSpec((_ROWS_BLK, D), lambda i: (i, 0)),
            pl.BlockSpec((_ROWS_BLK, D), lambda i: (i + _GRID, 0)),
            pl.BlockSpec((D, D), lambda i: (0, 0)),
            pl.BlockSpec((1, D), lambda i: (0, 0)),
            pl.BlockSpec((NW * L // D, D), lambda i: (0, 0)),
            pl.BlockSpec((NW * L // D, D), lambda i: (0, 0)),
        ],
        out_specs=[
            pl.BlockSpec((_ROWS_BLK, D), lambda i: (i, 0)),
            pl.BlockSpec((1, 1), lambda i: (0, 0)),
        ],
        out_shape=[
            jax.ShapeDtypeStruct((N, D), jnp.float32),
            jax.ShapeDtypeStruct((1, 1), jnp.float32),
        ],
    )(parts, parts, w, b, n1, n2)


def kernel(x, edge_index, W0, b0, W1, b1, a_mu, a_log_sigma,
           a_mu_first, a_log_sigma_first, eps_first, eps_rest):
    src = edge_index[0]
    dst = edge_index[1]
    sc_layer = _make_sc_layer()

    mu0 = a_mu_first.reshape(G, L)
    sig0 = a_log_sigma_first.reshape(G, L)
    part0, nllp0 = sc_layer(src, dst, x, eps_first, mu0, sig0)
    h = _tc_mid(part0, W0, b0.reshape(1, D))

    mu1 = a_mu[0].reshape(G, L)
    sig1 = a_log_sigma[0].reshape(G, L)
    eps1 = eps_rest.reshape(E, D)
    part1, nllp1 = sc_layer(src, dst, h, eps1, mu1, sig1)
    out, nll = _tc_final(part1, W1, b1.reshape(1, D),
                         nllp0.reshape(NW * L // D, D),
                         nllp1.reshape(NW * L // D, D))
    return (out, nll[0, 0])
